# baseline (speedup 1.0000x reference)
import jax
import jax.numpy as jnp
from jax.experimental import pallas as pl
from jax.experimental.pallas import tpu as pltpu


def _moe_mm_kernel(bias_ref, x_ref, w_ref, o_ref):
    x = x_ref[0]
    w = w_ref[0]
    acc = jax.lax.dot_general(
        x, w, (((1,), (1,)), ((), ())), preferred_element_type=jnp.float32
    )
    o_ref[0] = acc + bias_ref[0]


def kernel(inputs, moe_weight, expert_size):
    num_experts, output_size, input_size = moe_weight.shape
    total_tokens = inputs.shape[0]
    tokens_per_expert = total_tokens // num_experts

    x = inputs.reshape(num_experts, tokens_per_expert, input_size)
    bias = (
        jnp.asarray(expert_size, jnp.float32) - jnp.float32(tokens_per_expert)
    ).reshape(1)

    token_tile = 256
    num_token_tiles = tokens_per_expert // token_tile
    out = pl.pallas_call(
        _moe_mm_kernel,
        grid=(num_experts, num_token_tiles),
        in_specs=[
            pl.BlockSpec(memory_space=pltpu.SMEM),
            pl.BlockSpec((1, token_tile, input_size), lambda e, t: (e, t, 0)),
            pl.BlockSpec((1, output_size, input_size), lambda e, t: (e, 0, 0)),
        ],
        out_specs=pl.BlockSpec((1, token_tile, output_size), lambda e, t: (e, t, 0)),
        out_shape=jax.ShapeDtypeStruct(
            (num_experts, tokens_per_expert, output_size), jnp.float32
        ),
        compiler_params=pltpu.CompilerParams(
            dimension_semantics=("parallel", "arbitrary")
        ),
    )(bias, x, moe_weight)
    return out.reshape(total_tokens, output_size)


# speedup vs baseline: 1.7955x; 1.7955x over previous
import jax
import jax.numpy as jnp
from jax.experimental import pallas as pl
from jax.experimental.pallas import tpu as pltpu


def _moe_mm_kernel(bias_ref, x_ref, w_ref, o_ref):
    for i in range(x_ref.shape[0]):
        acc = jax.lax.dot_general(
            x_ref[i], w_ref[i], (((1,), (1,)), ((), ())),
            preferred_element_type=jnp.float32,
        )
        o_ref[i] = acc + bias_ref[0]


def kernel(inputs, moe_weight, expert_size):
    num_experts, output_size, input_size = moe_weight.shape
    total_tokens = inputs.shape[0]
    tokens_per_expert = total_tokens // num_experts

    x = inputs.reshape(num_experts, tokens_per_expert, input_size)
    bias = (
        jnp.asarray(expert_size, jnp.float32) - jnp.float32(tokens_per_expert)
    ).reshape(1)

    experts_per_step = 2
    num_steps = num_experts // experts_per_step
    out = pl.pallas_call(
        _moe_mm_kernel,
        grid=(num_steps,),
        in_specs=[
            pl.BlockSpec(memory_space=pltpu.SMEM),
            pl.BlockSpec(
                (experts_per_step, tokens_per_expert, input_size),
                lambda e: (e, 0, 0),
            ),
            pl.BlockSpec(
                (experts_per_step, output_size, input_size), lambda e: (e, 0, 0)
            ),
        ],
        out_specs=pl.BlockSpec(
            (experts_per_step, tokens_per_expert, output_size), lambda e: (e, 0, 0)
        ),
        out_shape=jax.ShapeDtypeStruct(
            (num_experts, tokens_per_expert, output_size), jnp.float32
        ),
        compiler_params=pltpu.CompilerParams(dimension_semantics=("parallel",)),
    )(bias, x, moe_weight)
    return out.reshape(total_tokens, output_size)


# speedup vs baseline: 1.8458x; 1.0280x over previous
import jax
import jax.numpy as jnp
from jax.experimental import pallas as pl
from jax.experimental.pallas import tpu as pltpu


def _moe_mm_kernel(bias_ref, x_ref, w_ref, o_ref):
    for i in range(x_ref.shape[0]):
        acc = jax.lax.dot_general(
            x_ref[i], w_ref[i], (((1,), (1,)), ((), ())),
            preferred_element_type=jnp.float32,
        )
        o_ref[i] = acc + bias_ref[0]


def kernel(inputs, moe_weight, expert_size):
    num_experts, output_size, input_size = moe_weight.shape
    total_tokens = inputs.shape[0]
    tokens_per_expert = total_tokens // num_experts

    x = inputs.reshape(num_experts, tokens_per_expert, input_size)
    bias = (
        jnp.asarray(expert_size, jnp.float32) - jnp.float32(tokens_per_expert)
    ).reshape(1)

    experts_per_step = 4
    num_steps = num_experts // experts_per_step
    out = pl.pallas_call(
        _moe_mm_kernel,
        grid=(num_steps,),
        in_specs=[
            pl.BlockSpec(memory_space=pltpu.SMEM),
            pl.BlockSpec(
                (experts_per_step, tokens_per_expert, input_size),
                lambda e: (e, 0, 0),
            ),
            pl.BlockSpec(
                (experts_per_step, output_size, input_size), lambda e: (e, 0, 0)
            ),
        ],
        out_specs=pl.BlockSpec(
            (experts_per_step, tokens_per_expert, output_size), lambda e: (e, 0, 0)
        ),
        out_shape=jax.ShapeDtypeStruct(
            (num_experts, tokens_per_expert, output_size), jnp.float32
        ),
        compiler_params=pltpu.CompilerParams(dimension_semantics=("parallel",)),
    )(bias, x, moe_weight)
    return out.reshape(total_tokens, output_size)
